# baseline (speedup 1.0000x reference)
import functools

import jax
import jax.numpy as jnp
from jax import lax
from jax.experimental import pallas as pl
from jax.experimental.pallas import tpu as pltpu
from jax.experimental.pallas import tpu_sc as plsc

HID = 64
K = 24
NB = 8


def _elu(x):
    return jnp.where(x > 0, x, jnp.exp(x) - 1.0)


def _enc_body(x_ref, w0, b0, w1, b1, w2, b2, o_ref):
    h = _elu(jnp.dot(x_ref[...], w0[...], preferred_element_type=jnp.float32)
             + b0[...])
    h = _elu(jnp.dot(h, w1[...], preferred_element_type=jnp.float32) + b1[...])
    o_ref[...] = (jnp.dot(h, w2[...], preferred_element_type=jnp.float32)
                  + b2[...])


def _enc(x_lc, enc_params, interpret=False):
    n = x_lc.shape[0]
    (w0, b0), (w1, b1), (w2, b2) = enc_params
    return pl.pallas_call(
        _enc_body,
        out_shape=jax.ShapeDtypeStruct((n, HID), jnp.float32),
        interpret=interpret,
    )(x_lc, w0, b0[None, :], w1, b1[None, :], w2, b2[None, :])


def _knn_body(x_ref, rows_ref, sqr_ref, sqc_ref, rs_ref, re_ref, seg_ref,
              idx_ref, slab, *, n, rblk, tw):
    b = pl.program_id(0)
    lo = seg_ref[0, b]
    hi = seg_ref[1, b]
    lo_al = lax.div(lo, tw) * tw
    ntiles = lax.div(hi - lo_al + (tw - 1), tw)

    rows = rows_ref[...]
    rows_sq = sqr_ref[...]
    rs = rs_ref[...]
    re = re_ref[...]
    lane = lax.broadcasted_iota(jnp.int32, (rblk, tw), 1)

    def fill(t, _):
        c0 = pl.multiple_of(jnp.minimum(lo_al + t * tw, n - tw), tw)
        cols = x_ref[pl.ds(c0, tw), :]
        mm = lax.dot_general(rows, cols, (((1,), (1,)), ((), ())),
                             preferred_element_type=jnp.float32)
        csq = sqc_ref[:, pl.ds(c0, tw)]
        d = (rows_sq - 2.0 * mm) + csq
        colg = lane + c0
        valid = (colg >= rs) & (colg < re)
        slab[t] = jnp.where(valid, d, jnp.inf)
        return 0

    lax.fori_loop(0, ntiles, fill, 0)

    mi = jnp.full((rblk, 1), -1, jnp.int32)
    picks = []
    for _ in range(K):
        prev = mi

        def scan(t, carry, prev=prev):
            bv, bi = carry
            c0 = pl.multiple_of(jnp.minimum(lo_al + t * tw, n - tw), tw)
            colg = lane + c0
            d = jnp.where(colg == prev, jnp.inf, slab[t])
            slab[t] = d
            tmin = jnp.min(d, axis=1, keepdims=True)
            targ = jnp.min(jnp.where(d == tmin, colg, n), axis=1,
                           keepdims=True)
            better = (tmin < bv) | ((tmin == bv) & (targ < bi))
            return (jnp.where(better, tmin, bv),
                    jnp.where(better, targ, bi))

        bv0 = jnp.full((rblk, 1), jnp.inf, jnp.float32)
        bi0 = jnp.full((rblk, 1), n, jnp.int32)
        _, mi = lax.fori_loop(0, ntiles, scan, (bv0, bi0))
        picks.append(mi)
    idx_ref[...] = jnp.concatenate(picks, axis=1)


def _knn(x, sq, rs, re, seg, rblk=256, tw=512, interpret=False):
    n, h = x.shape
    nblk = n // rblk
    ntmax = n // tw
    body = functools.partial(_knn_body, n=n, rblk=rblk, tw=tw)
    return pl.pallas_call(
        body,
        grid=(nblk,),
        in_specs=[
            pl.BlockSpec((n, h), lambda b: (0, 0)),
            pl.BlockSpec((rblk, h), lambda b: (b, 0)),
            pl.BlockSpec((rblk, 1), lambda b: (b, 0)),
            pl.BlockSpec((1, n), lambda b: (0, 0)),
            pl.BlockSpec((rblk, 1), lambda b: (b, 0)),
            pl.BlockSpec((rblk, 1), lambda b: (b, 0)),
            pl.BlockSpec(memory_space=pltpu.SMEM),
        ],
        out_specs=pl.BlockSpec((rblk, K), lambda b: (b, 0)),
        out_shape=jax.ShapeDtypeStruct((n, K), jnp.int32),
        scratch_shapes=[pltpu.VMEM((ntmax, rblk, tw), jnp.float32)],
        interpret=interpret,
    )(x, x, sq[:, None], sq[None, :], rs, re, seg)


def _gather_sc(table, idx3, per):
    nw, ch, _ = idx3.shape
    h = table.shape[1]
    mesh = plsc.VectorSubcoreMesh(core_axis_name="c", subcore_axis_name="s")

    @functools.partial(
        pl.kernel,
        mesh=mesh,
        out_type=jax.ShapeDtypeStruct((nw * per, h), jnp.float32),
        scratch_types=[
            pltpu.VMEM((ch, 128), jnp.int32),
            pltpu.VMEM((128, h), jnp.float32),
            pltpu.SemaphoreType.DMA,
        ],
    )
    def k(tab_hbm, idx_hbm, out_hbm, idxv, rowsv, sem):
        wid = lax.axis_index("s") * 2 + lax.axis_index("c")
        pltpu.sync_copy(idx_hbm.at[wid], idxv)

        def step(j, _):
            pltpu.async_copy(tab_hbm.at[idxv.at[j]], rowsv, sem).wait()
            pltpu.sync_copy(rowsv, out_hbm.at[pl.ds(wid * per + j * 128, 128)])
            return 0

        lax.fori_loop(0, ch, step, 0)

    return k(table, idx3)


def _tail_body(x_ref, xg_ref, w1, b1, w2, b2, gam, bet, o_ref, *, rblk):
    xv = x_ref[...]
    acc = jnp.full((rblk, HID), -jnp.inf, jnp.float32)
    for k in range(K):
        xj = xg_ref[:, k, :HID]
        m = jnp.concatenate([xv, xj - xv], axis=1)
        h = _elu(jnp.dot(m, w1[...], preferred_element_type=jnp.float32)
                 + b1[...])
        acc = jnp.maximum(
            acc, jnp.dot(h, w2[...], preferred_element_type=jnp.float32))
    y = _elu(acc + b2[...]) + xv
    mu = jnp.mean(y, axis=1, keepdims=True)
    var = jnp.mean((y - mu) ** 2, axis=1, keepdims=True)
    o_ref[...] = (y - mu) / jnp.sqrt(var + 1e-5) * gam[...] + bet[...]


def _tail(x, xg, w1, b1, w2, b2, gam, bet, rblk=256, interpret=False):
    n = x.shape[0]
    nblk = n // rblk
    body = functools.partial(_tail_body, rblk=rblk)
    return pl.pallas_call(
        body,
        grid=(nblk,),
        in_specs=[
            pl.BlockSpec((rblk, HID), lambda b: (b, 0)),
            pl.BlockSpec((rblk, K, 128), lambda b: (b, 0, 0)),
            pl.BlockSpec((2 * HID, HID), lambda b: (0, 0)),
            pl.BlockSpec((1, HID), lambda b: (0, 0)),
            pl.BlockSpec((HID, HID), lambda b: (0, 0)),
            pl.BlockSpec((1, HID), lambda b: (0, 0)),
            pl.BlockSpec((1, HID), lambda b: (0, 0)),
            pl.BlockSpec((1, HID), lambda b: (0, 0)),
        ],
        out_specs=pl.BlockSpec((rblk, HID), lambda b: (b, 0)),
        out_shape=jax.ShapeDtypeStruct((n, HID), jnp.float32),
        interpret=interpret,
    )(x, xg, w1, b1[None, :], w2, b2[None, :], gam[None, :], bet[None, :])


def _out_body(x_ref, wa, ba, wb, bb, wc, bc, o_ref):
    h = _elu(jnp.dot(x_ref[...], wa[...], preferred_element_type=jnp.float32)
             + ba[...])
    h = _elu(jnp.dot(h, wb[...], preferred_element_type=jnp.float32) + bb[...])
    o_ref[...] = (jnp.dot(h, wc[...], preferred_element_type=jnp.float32)
                  + bc[...])


def _outmlp(x, out_params, interpret=False):
    n = x.shape[0]
    (wa, ba), (wb, bb), (wc, bc) = out_params
    return pl.pallas_call(
        _out_body,
        out_shape=jax.ShapeDtypeStruct((n, wc.shape[1]), jnp.float32),
        interpret=interpret,
    )(x, wa, ba[None, :], wb, bb[None, :], wc, bc[None, :])


def kernel(x_lc, params, batch_lc):
    n = x_lc.shape[0]
    rblk = 1024
    tw = 1024
    batch = batch_lc.astype(jnp.int32)

    ar = jnp.arange(NB, dtype=jnp.int32)
    starts = jnp.searchsorted(batch, ar, side="left").astype(jnp.int32)
    ends = jnp.searchsorted(batch, ar, side="right").astype(jnp.int32)
    rs = starts[batch][:, None]
    re = ends[batch][:, None]
    sizes = ends - starts
    fb = batch[0::rblk]
    lb = batch[rblk - 1::rblk]
    lo = starts[fb]
    hi = ends[lb]
    in_rng = (ar[None, :] >= fb[:, None]) & (ar[None, :] <= lb[:, None])
    msize = jnp.min(jnp.where(in_rng, sizes[None, :], n), axis=1)
    small = msize < K
    lo = jnp.where(small, 0, lo)
    hi = jnp.where(small, n, hi)
    seg = jnp.stack([lo, hi]).astype(jnp.int32)

    x = _enc(x_lc, params["enc"])
    for i in (1, 2, 3):
        (w1, b1), (w2, b2) = params["conv%d" % i]
        sq = jnp.sum(x * x, axis=1)
        idx = _knn(x, sq, rs, re, seg, rblk=rblk, tw=tw)
        nw = 32
        per = (n * K) // nw
        idx3 = idx.reshape(nw, per // 128, 128)
        x_pad = jnp.pad(x, ((0, 0), (0, 128 - HID)))
        xg = _gather_sc(x_pad, idx3, per).reshape(n, K, 128)
        gam, bet = params["norm%d" % i]
        x = _tail(x, xg, w1, b1, w2, b2, gam, bet, rblk=256)
    out = _outmlp(x, params["out"])
    return (out, batch_lc)


# speedup vs baseline: 1.2850x; 1.2850x over previous
import functools

import jax
import jax.numpy as jnp
from jax import lax
from jax.experimental import pallas as pl
from jax.experimental.pallas import tpu as pltpu
from jax.experimental.pallas import tpu_sc as plsc

HID = 64
K = 24
NB = 8


def _elu(x):
    return jnp.where(x > 0, x, jnp.exp(x) - 1.0)


def _enc_body(x_ref, w0, b0, w1, b1, w2, b2, o_ref):
    h = _elu(jnp.dot(x_ref[...], w0[...], preferred_element_type=jnp.float32)
             + b0[...])
    h = _elu(jnp.dot(h, w1[...], preferred_element_type=jnp.float32) + b1[...])
    o_ref[...] = (jnp.dot(h, w2[...], preferred_element_type=jnp.float32)
                  + b2[...])


def _enc(x_lc, enc_params, interpret=False):
    n = x_lc.shape[0]
    (w0, b0), (w1, b1), (w2, b2) = enc_params
    return pl.pallas_call(
        _enc_body,
        out_shape=jax.ShapeDtypeStruct((n, HID), jnp.float32),
        interpret=interpret,
    )(x_lc, w0, b0[None, :], w1, b1[None, :], w2, b2[None, :])


def _knn_body(x_ref, rows_ref, sqr_ref, sqc_ref, rs_ref, re_ref, seg_ref,
              idx_ref, slab, *, n, rblk, tw):
    b = pl.program_id(0)
    lo = seg_ref[0, b]
    hi = seg_ref[1, b]
    lo_al = lax.div(lo, tw) * tw
    ntiles = lax.div(hi - lo_al + (tw - 1), tw)

    rows = rows_ref[...]
    rows_sq = sqr_ref[...]
    rs = rs_ref[...]
    re = re_ref[...]
    lane = lax.broadcasted_iota(jnp.int32, (rblk, tw), 1)

    def fill(t, _):
        c0 = pl.multiple_of(jnp.minimum(lo_al + t * tw, n - tw), tw)
        cols = x_ref[pl.ds(c0, tw), :]
        mm = lax.dot_general(rows, cols, (((1,), (1,)), ((), ())),
                             preferred_element_type=jnp.float32)
        csq = sqc_ref[:, pl.ds(c0, tw)]
        d = (rows_sq - 2.0 * mm) + csq
        colg = lane + c0
        valid = (colg >= rs) & (colg < re)
        slab[t] = jnp.where(valid, d, jnp.inf)
        return 0

    lax.fori_loop(0, ntiles, fill, 0)

    mi = jnp.full((rblk, 1), -1, jnp.int32)
    picks = []
    for _ in range(K):
        prev = mi

        def scan(t, carry, prev=prev):
            bv, bi = carry
            c0 = pl.multiple_of(jnp.minimum(lo_al + t * tw, n - tw), tw)
            colg = lane + c0
            d = jnp.where(colg == prev, jnp.inf, slab[t])
            slab[t] = d
            tmin = jnp.min(d, axis=1, keepdims=True)
            targ = jnp.min(jnp.where(d == tmin, colg, n), axis=1,
                           keepdims=True)
            better = (tmin < bv) | ((tmin == bv) & (targ < bi))
            return (jnp.where(better, tmin, bv),
                    jnp.where(better, targ, bi))

        bv0 = jnp.full((rblk, 1), jnp.inf, jnp.float32)
        bi0 = jnp.full((rblk, 1), n, jnp.int32)
        _, mi = lax.fori_loop(0, ntiles, scan, (bv0, bi0))
        picks.append(mi)
    idx_ref[...] = jnp.concatenate(picks, axis=1)


def _knn(x, sq, rs, re, seg, rblk=256, tw=512, interpret=False):
    n, h = x.shape
    nblk = n // rblk
    ntmax = n // tw
    body = functools.partial(_knn_body, n=n, rblk=rblk, tw=tw)
    return pl.pallas_call(
        body,
        grid=(nblk,),
        in_specs=[
            pl.BlockSpec((n, h), lambda b: (0, 0)),
            pl.BlockSpec((rblk, h), lambda b: (b, 0)),
            pl.BlockSpec((rblk, 1), lambda b: (b, 0)),
            pl.BlockSpec((1, n), lambda b: (0, 0)),
            pl.BlockSpec((rblk, 1), lambda b: (b, 0)),
            pl.BlockSpec((rblk, 1), lambda b: (b, 0)),
            pl.BlockSpec(memory_space=pltpu.SMEM),
        ],
        out_specs=pl.BlockSpec((rblk, K), lambda b: (b, 0)),
        out_shape=jax.ShapeDtypeStruct((n, K), jnp.int32),
        scratch_shapes=[pltpu.VMEM((ntmax, rblk, tw), jnp.float32)],
        interpret=interpret,
    )(x, x, sq[:, None], sq[None, :], rs, re, seg)


def _gather_sc(table, idx3, per):
    nw, ch, _ = idx3.shape
    h = table.shape[1]
    mesh = plsc.VectorSubcoreMesh(core_axis_name="c", subcore_axis_name="s")

    @functools.partial(
        pl.kernel,
        mesh=mesh,
        out_type=jax.ShapeDtypeStruct((nw * per, h), jnp.float32),
        scratch_types=[
            pltpu.VMEM((ch, 128), jnp.int32),
            pltpu.VMEM((128, h), jnp.float32),
            pltpu.VMEM((128, h), jnp.float32),
            pltpu.SemaphoreType.DMA,
            pltpu.SemaphoreType.DMA,
        ],
    )
    def k(tab_hbm, idx_hbm, out_hbm, idxv, rows0, rows1, sem0, sem1):
        wid = lax.axis_index("s") * 2 + lax.axis_index("c")
        base = wid * per
        pltpu.sync_copy(idx_hbm.at[wid], idxv)
        pltpu.async_copy(tab_hbm.at[idxv.at[0]], rows0, sem0)

        def step(p, _):
            j0 = 2 * p
            pltpu.async_copy(tab_hbm.at[idxv.at[j0 + 1]], rows1, sem1)
            pltpu.make_async_copy(tab_hbm.at[idxv.at[j0]], rows0, sem0).wait()
            pltpu.sync_copy(rows0, out_hbm.at[pl.ds(base + j0 * 128, 128)])

            @pl.when(p + 1 < ch // 2)
            def _():
                pltpu.async_copy(tab_hbm.at[idxv.at[j0 + 2]], rows0, sem0)

            pltpu.make_async_copy(tab_hbm.at[idxv.at[j0 + 1]], rows1,
                                  sem1).wait()
            pltpu.sync_copy(rows1,
                            out_hbm.at[pl.ds(base + (j0 + 1) * 128, 128)])
            return 0

        lax.fori_loop(0, ch // 2, step, 0)

    return k(table, idx3)


def _tail_body(x_ref, xg_ref, w1, b1, w2, b2, gam, bet, o_ref, *, rblk):
    xv = x_ref[...]
    acc = jnp.full((rblk, HID), -jnp.inf, jnp.float32)
    for k in range(K):
        xj = xg_ref[:, k, :HID]
        m = jnp.concatenate([xv, xj - xv], axis=1)
        h = _elu(jnp.dot(m, w1[...], preferred_element_type=jnp.float32)
                 + b1[...])
        acc = jnp.maximum(
            acc, jnp.dot(h, w2[...], preferred_element_type=jnp.float32))
    y = _elu(acc + b2[...]) + xv
    mu = jnp.mean(y, axis=1, keepdims=True)
    var = jnp.mean((y - mu) ** 2, axis=1, keepdims=True)
    o_ref[...] = (y - mu) / jnp.sqrt(var + 1e-5) * gam[...] + bet[...]


def _tail(x, xg, w1, b1, w2, b2, gam, bet, rblk=256, interpret=False):
    n = x.shape[0]
    nblk = n // rblk
    body = functools.partial(_tail_body, rblk=rblk)
    return pl.pallas_call(
        body,
        grid=(nblk,),
        in_specs=[
            pl.BlockSpec((rblk, HID), lambda b: (b, 0)),
            pl.BlockSpec((rblk, K, 128), lambda b: (b, 0, 0)),
            pl.BlockSpec((2 * HID, HID), lambda b: (0, 0)),
            pl.BlockSpec((1, HID), lambda b: (0, 0)),
            pl.BlockSpec((HID, HID), lambda b: (0, 0)),
            pl.BlockSpec((1, HID), lambda b: (0, 0)),
            pl.BlockSpec((1, HID), lambda b: (0, 0)),
            pl.BlockSpec((1, HID), lambda b: (0, 0)),
        ],
        out_specs=pl.BlockSpec((rblk, HID), lambda b: (b, 0)),
        out_shape=jax.ShapeDtypeStruct((n, HID), jnp.float32),
        interpret=interpret,
    )(x, xg, w1, b1[None, :], w2, b2[None, :], gam[None, :], bet[None, :])


def _out_body(x_ref, wa, ba, wb, bb, wc, bc, o_ref):
    h = _elu(jnp.dot(x_ref[...], wa[...], preferred_element_type=jnp.float32)
             + ba[...])
    h = _elu(jnp.dot(h, wb[...], preferred_element_type=jnp.float32) + bb[...])
    o_ref[...] = (jnp.dot(h, wc[...], preferred_element_type=jnp.float32)
                  + bc[...])


def _outmlp(x, out_params, interpret=False):
    n = x.shape[0]
    (wa, ba), (wb, bb), (wc, bc) = out_params
    return pl.pallas_call(
        _out_body,
        out_shape=jax.ShapeDtypeStruct((n, wc.shape[1]), jnp.float32),
        interpret=interpret,
    )(x, wa, ba[None, :], wb, bb[None, :], wc, bc[None, :])


def kernel(x_lc, params, batch_lc):
    n = x_lc.shape[0]
    rblk = 512
    tw = 1024
    batch = batch_lc.astype(jnp.int32)

    ar = jnp.arange(NB, dtype=jnp.int32)
    starts = jnp.searchsorted(batch, ar, side="left").astype(jnp.int32)
    ends = jnp.searchsorted(batch, ar, side="right").astype(jnp.int32)
    rs = starts[batch][:, None]
    re = ends[batch][:, None]
    sizes = ends - starts
    fb = batch[0::rblk]
    lb = batch[rblk - 1::rblk]
    lo = starts[fb]
    hi = ends[lb]
    in_rng = (ar[None, :] >= fb[:, None]) & (ar[None, :] <= lb[:, None])
    msize = jnp.min(jnp.where(in_rng, sizes[None, :], n), axis=1)
    small = msize < K
    lo = jnp.where(small, 0, lo)
    hi = jnp.where(small, n, hi)
    seg = jnp.stack([lo, hi]).astype(jnp.int32)

    x = _enc(x_lc, params["enc"])
    for i in (1, 2, 3):
        (w1, b1), (w2, b2) = params["conv%d" % i]
        sq = jnp.sum(x * x, axis=1)
        idx = _knn(x, sq, rs, re, seg, rblk=rblk, tw=tw)
        nw = 32
        per = (n * K) // nw
        idx3 = idx.reshape(nw, per // 128, 128)
        x_pad = jnp.pad(x, ((0, 0), (0, 128 - HID)))
        xg = _gather_sc(x_pad, idx3, per).reshape(n, K, 128)
        gam, bet = params["norm%d" % i]
        x = _tail(x, xg, w1, b1, w2, b2, gam, bet, rblk=256)
    out = _outmlp(x, params["out"])
    return (out, batch_lc)


# speedup vs baseline: 1.6171x; 1.2585x over previous
import functools

import jax
import jax.numpy as jnp
from jax import lax
from jax.experimental import pallas as pl
from jax.experimental.pallas import tpu as pltpu
from jax.experimental.pallas import tpu_sc as plsc

HID = 64
K = 24
NB = 8


def _elu(x):
    return jnp.where(x > 0, x, jnp.exp(x) - 1.0)


def _enc_body(x_ref, w0, b0, w1, b1, w2, b2, o_ref):
    h = _elu(jnp.dot(x_ref[...], w0[...], preferred_element_type=jnp.float32)
             + b0[...])
    h = _elu(jnp.dot(h, w1[...], preferred_element_type=jnp.float32) + b1[...])
    o_ref[...] = (jnp.dot(h, w2[...], preferred_element_type=jnp.float32)
                  + b2[...])


def _enc(x_lc, enc_params, interpret=False):
    n = x_lc.shape[0]
    (w0, b0), (w1, b1), (w2, b2) = enc_params
    return pl.pallas_call(
        _enc_body,
        out_shape=jax.ShapeDtypeStruct((n, HID), jnp.float32),
        interpret=interpret,
    )(x_lc, w0, b0[None, :], w1, b1[None, :], w2, b2[None, :])


def _knn_body(x_ref, rows_ref, sqr_ref, sqc_ref, rs_ref, re_ref, seg_ref,
              idx_ref, slab, *, n, rblk, tw):
    b = pl.program_id(0)
    lo = seg_ref[0, b]
    hi = seg_ref[1, b]
    lo_al = lax.div(lo, tw) * tw
    ntiles = lax.div(hi - lo_al + (tw - 1), tw)

    rows = rows_ref[...]
    rows_sq = sqr_ref[...]
    rs = rs_ref[...]
    re = re_ref[...]
    lane = lax.broadcasted_iota(jnp.int32, (rblk, tw), 1)
    lane_f = lane.astype(jnp.float32)

    def fill(t, _):
        c0 = pl.multiple_of(jnp.minimum(lo_al + t * tw, n - tw), tw)
        cols = x_ref[pl.ds(c0, tw), :]
        mm = lax.dot_general(rows, cols, (((1,), (1,)), ((), ())),
                             preferred_element_type=jnp.float32)
        csq = sqc_ref[:, pl.ds(c0, tw)]
        d = (rows_sq - 2.0 * mm) + csq
        colg = lane + c0
        valid = (colg >= rs) & (colg < re)
        slab[t] = jnp.where(valid, d, jnp.inf)
        return 0

    lax.fori_loop(0, ntiles, fill, 0)

    mi = jnp.full((rblk, 1), -1.0, jnp.float32)
    picks = []
    for _ in range(K):
        prev = mi

        def scan(t, carry, prev=prev):
            bv, bi = carry
            c0 = pl.multiple_of(jnp.minimum(lo_al + t * tw, n - tw), tw)
            colg = lane_f + c0.astype(jnp.float32)
            d = jnp.where(colg == prev, jnp.inf, slab[t])
            slab[t] = d
            tmin = jnp.min(d, axis=1, keepdims=True)
            targ = jnp.min(jnp.where(d == tmin, colg, float(n)), axis=1,
                           keepdims=True)
            better = (tmin < bv) | ((tmin == bv) & (targ < bi))
            return (jnp.where(better, tmin, bv),
                    jnp.where(better, targ, bi))

        bv0 = jnp.full((rblk, 1), jnp.inf, jnp.float32)
        bi0 = jnp.full((rblk, 1), float(n), jnp.float32)
        _, mi = lax.fori_loop(0, ntiles, scan, (bv0, bi0))
        picks.append(mi)
    idx_ref[...] = jnp.concatenate(picks, axis=1).astype(jnp.int32)


def _knn(x, sq, rs, re, seg, rblk=256, tw=512, interpret=False):
    n, h = x.shape
    nblk = n // rblk
    ntmax = n // tw
    body = functools.partial(_knn_body, n=n, rblk=rblk, tw=tw)
    return pl.pallas_call(
        body,
        grid=(nblk,),
        in_specs=[
            pl.BlockSpec((n, h), lambda b: (0, 0)),
            pl.BlockSpec((rblk, h), lambda b: (b, 0)),
            pl.BlockSpec((rblk, 1), lambda b: (b, 0)),
            pl.BlockSpec((1, n), lambda b: (0, 0)),
            pl.BlockSpec((rblk, 1), lambda b: (b, 0)),
            pl.BlockSpec((rblk, 1), lambda b: (b, 0)),
            pl.BlockSpec(memory_space=pltpu.SMEM),
        ],
        out_specs=pl.BlockSpec((rblk, K), lambda b: (b, 0)),
        out_shape=jax.ShapeDtypeStruct((n, K), jnp.int32),
        scratch_shapes=[pltpu.VMEM((ntmax, rblk, tw), jnp.float32)],
        interpret=interpret,
    )(x, x, sq[:, None], sq[None, :], rs, re, seg)


def _gather_sc(table, idx3, per):
    nw, ch, _ = idx3.shape
    h = table.shape[1]
    mesh = plsc.VectorSubcoreMesh(core_axis_name="c", subcore_axis_name="s")

    @functools.partial(
        pl.kernel,
        mesh=mesh,
        out_type=jax.ShapeDtypeStruct((nw * per, h), jnp.float32),
        scratch_types=[
            pltpu.VMEM((ch, 128), jnp.int32),
            pltpu.VMEM((128, h), jnp.float32),
            pltpu.VMEM((128, h), jnp.float32),
            pltpu.SemaphoreType.DMA,
            pltpu.SemaphoreType.DMA,
        ],
    )
    def k(tab_hbm, idx_hbm, out_hbm, idxv, rows0, rows1, sem0, sem1):
        wid = lax.axis_index("s") * 2 + lax.axis_index("c")
        base = wid * per
        pltpu.sync_copy(idx_hbm.at[wid], idxv)
        pltpu.async_copy(tab_hbm.at[idxv.at[0]], rows0, sem0)

        def step(p, _):
            j0 = 2 * p
            pltpu.async_copy(tab_hbm.at[idxv.at[j0 + 1]], rows1, sem1)
            pltpu.make_async_copy(tab_hbm.at[idxv.at[j0]], rows0, sem0).wait()
            pltpu.sync_copy(rows0, out_hbm.at[pl.ds(base + j0 * 128, 128)])

            @pl.when(p + 1 < ch // 2)
            def _():
                pltpu.async_copy(tab_hbm.at[idxv.at[j0 + 2]], rows0, sem0)

            pltpu.make_async_copy(tab_hbm.at[idxv.at[j0 + 1]], rows1,
                                  sem1).wait()
            pltpu.sync_copy(rows1,
                            out_hbm.at[pl.ds(base + (j0 + 1) * 128, 128)])
            return 0

        lax.fori_loop(0, ch // 2, step, 0)

    return k(table, idx3)


def _tail_body(x_ref, xg_ref, w1, b1, w2, b2, gam, bet, o_ref, *, rblk):
    xv = x_ref[...]
    acc = jnp.full((rblk, HID), -jnp.inf, jnp.float32)
    for k in range(K):
        xj = xg_ref[k, :, :HID]
        m = jnp.concatenate([xv, xj - xv], axis=1)
        h = _elu(jnp.dot(m, w1[...], preferred_element_type=jnp.float32)
                 + b1[...])
        acc = jnp.maximum(
            acc, jnp.dot(h, w2[...], preferred_element_type=jnp.float32))
    y = _elu(acc + b2[...]) + xv
    mu = jnp.mean(y, axis=1, keepdims=True)
    var = jnp.mean((y - mu) ** 2, axis=1, keepdims=True)
    o_ref[...] = (y - mu) / jnp.sqrt(var + 1e-5) * gam[...] + bet[...]


def _tail(x, xg, w1, b1, w2, b2, gam, bet, rblk=256, interpret=False):
    n = x.shape[0]
    nblk = n // rblk
    body = functools.partial(_tail_body, rblk=rblk)
    return pl.pallas_call(
        body,
        grid=(nblk,),
        in_specs=[
            pl.BlockSpec((rblk, HID), lambda b: (b, 0)),
            pl.BlockSpec((K, rblk, 128), lambda b: (0, b, 0)),
            pl.BlockSpec((2 * HID, HID), lambda b: (0, 0)),
            pl.BlockSpec((1, HID), lambda b: (0, 0)),
            pl.BlockSpec((HID, HID), lambda b: (0, 0)),
            pl.BlockSpec((1, HID), lambda b: (0, 0)),
            pl.BlockSpec((1, HID), lambda b: (0, 0)),
            pl.BlockSpec((1, HID), lambda b: (0, 0)),
        ],
        out_specs=pl.BlockSpec((rblk, HID), lambda b: (b, 0)),
        out_shape=jax.ShapeDtypeStruct((n, HID), jnp.float32),
        interpret=interpret,
    )(x, xg, w1, b1[None, :], w2, b2[None, :], gam[None, :], bet[None, :])


def _out_body(x_ref, wa, ba, wb, bb, wc, bc, o_ref):
    h = _elu(jnp.dot(x_ref[...], wa[...], preferred_element_type=jnp.float32)
             + ba[...])
    h = _elu(jnp.dot(h, wb[...], preferred_element_type=jnp.float32) + bb[...])
    o_ref[...] = (jnp.dot(h, wc[...], preferred_element_type=jnp.float32)
                  + bc[...])


def _outmlp(x, out_params, interpret=False):
    n = x.shape[0]
    (wa, ba), (wb, bb), (wc, bc) = out_params
    return pl.pallas_call(
        _out_body,
        out_shape=jax.ShapeDtypeStruct((n, wc.shape[1]), jnp.float32),
        interpret=interpret,
    )(x, wa, ba[None, :], wb, bb[None, :], wc, bc[None, :])


def kernel(x_lc, params, batch_lc):
    n = x_lc.shape[0]
    rblk = 512
    tw = 1024
    batch = batch_lc.astype(jnp.int32)

    ar = jnp.arange(NB, dtype=jnp.int32)
    starts = jnp.searchsorted(batch, ar, side="left").astype(jnp.int32)
    ends = jnp.searchsorted(batch, ar, side="right").astype(jnp.int32)
    rs = starts[batch][:, None]
    re = ends[batch][:, None]
    sizes = ends - starts
    fb = batch[0::rblk]
    lb = batch[rblk - 1::rblk]
    lo = starts[fb]
    hi = ends[lb]
    in_rng = (ar[None, :] >= fb[:, None]) & (ar[None, :] <= lb[:, None])
    msize = jnp.min(jnp.where(in_rng, sizes[None, :], n), axis=1)
    small = msize < K
    lo = jnp.where(small, 0, lo)
    hi = jnp.where(small, n, hi)
    seg = jnp.stack([lo, hi]).astype(jnp.int32)

    x = _enc(x_lc, params["enc"])
    for i in (1, 2, 3):
        (w1, b1), (w2, b2) = params["conv%d" % i]
        sq = jnp.sum(x * x, axis=1)
        idx = _knn(x, sq, rs, re, seg, rblk=rblk, tw=tw)
        nw = 32
        per = (n * K) // nw
        idx3 = idx.T.reshape(nw, per // 128, 128)
        x_pad = jnp.pad(x, ((0, 0), (0, 128 - HID)))
        xg = _gather_sc(x_pad, idx3, per).reshape(K, n, 128)
        gam, bet = params["norm%d" % i]
        x = _tail(x, xg, w1, b1, w2, b2, gam, bet, rblk=256)
    out = _outmlp(x, params["out"])
    return (out, batch_lc)
